# baseline (speedup 1.0000x reference)
import functools

import jax
import jax.numpy as jnp
from jax import lax
from jax.experimental import pallas as pl
from jax.experimental.pallas import tpu as pltpu
from jax.experimental.pallas import tpu_sc as plsc

H = 128
HG = H // 16
N = 10000
EDGES = 320000
ELBL = 131072
NCORES = 2
NSUB = 16
NW = NCORES * NSUB

EC = EDGES // NSUB
CH = 80
IBLK = 20
SBLK = CH * IBLK
NSB = EC // SBLK
MCH = 80
NMCH = N // MCH

DCH = 800
DNCHUNK = EC // DCH
DMC = 400
DNMC = N // DMC

CC = 256
EPW = ELBL // NW
NCC = EPW // CC

_mesh = plsc.VectorSubcoreMesh(core_axis_name="c", subcore_axis_name="s")


def _agg_body(want_deg, *refs):
    if want_deg:
        (table, src_idx, dst_idx, mean_out, inv_out,
         acc, deg_sp, idxs, idxd, ones_v, d_v, rows0, rows1,
         gsem0, gsem1, ssem0, ssem1) = refs
    else:
        (table, src_idx, dst_idx, inv_deg, mean_out,
         acc, idxs, idxd, d_v, rows0, rows1,
         gsem0, gsem1, ssem0, ssem1) = refs
    c = lax.axis_index("c")
    s = lax.axis_index("s")
    nmine = (NMCH - s + NSUB - 1) // NSUB
    z16v = jnp.zeros((16,), jnp.float32)

    def fill0(r, carry):
        for h in range(HG):
            rows0[r, pl.ds(h * 16, 16)] = z16v
        return carry

    lax.fori_loop(0, MCH, fill0, 0)
    if want_deg:
        one16 = jnp.full((16,), 1.0, jnp.float32)

        def fillz1(g, carry):
            ones_v[pl.ds(g * 16, 16)] = z16v
            return carry

        lax.fori_loop(0, SBLK // 16, fillz1, 0)

    def zchunk(k, carry):
        row0 = (s + k * NSUB) * MCH
        pltpu.sync_copy(rows0, acc.at[pl.ds(row0, MCH)])
        if want_deg:
            pltpu.sync_copy(ones_v.at[pl.ds(0, MCH)], deg_sp.at[pl.ds(row0, MCH)])
        return carry

    lax.fori_loop(0, nmine, zchunk, 0)
    if want_deg:
        def fillo(g, carry):
            ones_v[pl.ds(g * 16, 16)] = one16
            return carry

        lax.fori_loop(0, SBLK // 16, fillo, 0)
    plsc.subcore_barrier()

    rowbufs = (rows0, rows1)
    gsems = (gsem0, gsem1)
    ssems = (ssem0, ssem1)

    def sblock(j, carry):
        base = s * EC + j * SBLK
        pltpu.sync_copy(src_idx.at[pl.ds(c * EDGES + base, SBLK)], idxs)
        pltpu.sync_copy(dst_idx.at[pl.ds(c * EDGES + base, SBLK)], idxd)
        if want_deg:
            pltpu.sync_copy(ones_v, deg_sp.at[idxd], add=True)
        pltpu.async_copy(table.at[idxs.at[pl.ds(0, CH)]], rows0, gsems[0])
        for i in range(IBLK):
            b = i % 2
            nb = (i + 1) % 2
            pltpu.make_async_copy(
                table.at[idxs.at[pl.ds(i * CH, CH)]],
                rowbufs[b], gsems[b]).wait()
            pltpu.async_copy(rowbufs[b], acc.at[idxd.at[pl.ds(i * CH, CH)]],
                             ssems[b], add=True)
            if i + 1 < IBLK:
                if i >= 1:
                    pltpu.make_async_copy(
                        rowbufs[nb],
                        acc.at[idxd.at[pl.ds((i - 1) * CH, CH)]],
                        ssems[nb]).wait()
                pltpu.async_copy(
                    table.at[idxs.at[pl.ds((i + 1) * CH, CH)]],
                    rowbufs[nb], gsems[nb])
        pltpu.make_async_copy(
            rowbufs[(IBLK - 2) % 2],
            acc.at[idxd.at[pl.ds((IBLK - 2) * CH, CH)]],
            ssems[(IBLK - 2) % 2]).wait()
        pltpu.make_async_copy(
            rowbufs[(IBLK - 1) % 2],
            acc.at[idxd.at[pl.ds((IBLK - 1) * CH, CH)]],
            ssems[(IBLK - 1) % 2]).wait()
        return carry

    lax.fori_loop(0, NSB, sblock, 0)
    plsc.subcore_barrier()

    def mchunk(k, carry):
        row0 = (s + k * NSUB) * MCH
        pltpu.sync_copy(acc.at[pl.ds(row0, MCH)], rows0)
        if want_deg:
            pltpu.sync_copy(deg_sp.at[pl.ds(row0, MCH)], d_v)

            def igrp(g, carry2):
                v = d_v[pl.ds(g * 16, 16)]
                d_v[pl.ds(g * 16, 16)] = 1.0 / jnp.maximum(v, 1.0)
                return carry2

            lax.fori_loop(0, MCH // 16, igrp, 0)
            pltpu.sync_copy(d_v, inv_out.at[pl.ds(c * N + row0, MCH)])
        else:
            pltpu.sync_copy(inv_deg.at[pl.ds(c * N + row0, MCH)], d_v)

        def mgrp(g, carry2):
            invv = d_v[pl.ds(g * 16, 16)]
            for i in range(16):
                inv = invv[i]
                for h in range(HG):
                    rows0[g * 16 + i, pl.ds(h * 16, 16)] = (
                        rows0[g * 16 + i, pl.ds(h * 16, 16)] * inv)
            return carry2

        lax.fori_loop(0, MCH // 16, mgrp, 0)
        pltpu.sync_copy(rows0, mean_out.at[c, pl.ds(row0, MCH)])
        return carry

    lax.fori_loop(0, nmine, mchunk, 0)


_agg1 = pl.kernel(
    functools.partial(_agg_body, True),
    mesh=_mesh,
    out_type=(jax.ShapeDtypeStruct((NCORES, N, H), jnp.float32),
              jax.ShapeDtypeStruct((NCORES * N,), jnp.float32)),
    scratch_types=[
        pltpu.VMEM_SHARED((N, H), jnp.float32),
        pltpu.VMEM_SHARED((N,), jnp.float32),
        pltpu.VMEM((SBLK,), jnp.int32),
        pltpu.VMEM((SBLK,), jnp.int32),
        pltpu.VMEM((SBLK,), jnp.float32),
        pltpu.VMEM((MCH,), jnp.float32),
        pltpu.VMEM((CH, H), jnp.float32),
        pltpu.VMEM((CH, H), jnp.float32),
        pltpu.SemaphoreType.DMA,
        pltpu.SemaphoreType.DMA,
        pltpu.SemaphoreType.DMA,
        pltpu.SemaphoreType.DMA,
    ],
)

_agg2 = pl.kernel(
    functools.partial(_agg_body, False),
    mesh=_mesh,
    out_type=jax.ShapeDtypeStruct((NCORES, N, H), jnp.float32),
    scratch_types=[
        pltpu.VMEM_SHARED((N, H), jnp.float32),
        pltpu.VMEM((SBLK,), jnp.int32),
        pltpu.VMEM((SBLK,), jnp.int32),
        pltpu.VMEM((MCH,), jnp.float32),
        pltpu.VMEM((CH, H), jnp.float32),
        pltpu.VMEM((CH, H), jnp.float32),
        pltpu.SemaphoreType.DMA,
        pltpu.SemaphoreType.DMA,
        pltpu.SemaphoreType.DMA,
        pltpu.SemaphoreType.DMA,
    ],
)


def _transform_body(relu, mean_ref, root_ref, wn_ref, wr_ref, b_ref, out_ref):
    o = (jnp.dot(mean_ref[0], wn_ref[0], preferred_element_type=jnp.float32)
         + jnp.dot(root_ref[0], wr_ref[0], preferred_element_type=jnp.float32)
         + b_ref[0])
    if relu:
        o = jnp.maximum(o, 0.0)
    out_ref[0] = o


def _make_transform(relu):
    RB = 1000
    return pl.pallas_call(
        functools.partial(_transform_body, relu),
        grid=(2, N // RB),
        in_specs=[
            pl.BlockSpec((1, RB, H), lambda j, i: (j, i, 0)),
            pl.BlockSpec((1, RB, H), lambda j, i: (1 - j, i, 0)),
            pl.BlockSpec((1, H, H), lambda j, i: (j, 0, 0)),
            pl.BlockSpec((1, H, H), lambda j, i: (j, 0, 0)),
            pl.BlockSpec((1, 1, H), lambda j, i: (j, 0, 0)),
        ],
        out_specs=pl.BlockSpec((1, RB, H), lambda j, i: (1 - j, i, 0)),
        out_shape=jax.ShapeDtypeStruct((2, N, H), jnp.float32),
    )


_transform_relu = _make_transform(True)
_transform_lin = _make_transform(False)


HC = CC // 2


def _cls_body(ot, el_idx, out, ia, ib, a_v, b_v, p_v, idx_p, z_v, o_sp,
              sa0, sb0, sa1, sb1):
    c = lax.axis_index("c")
    s = lax.axis_index("s")
    w = s * NCORES + c
    zero16 = jnp.zeros((16,), jnp.float32)

    def fillidx(g, carry):
        idx_p[pl.ds(g * 16, 16)] = jnp.full((16,), s * CC + g, jnp.int32)
        return carry

    lax.fori_loop(0, CC, fillidx, 0)

    def fillz(g, carry):
        z_v[pl.ds(g * 16, 16)] = zero16
        return carry

    lax.fori_loop(0, CC // 16, fillz, 0)

    pltpu.sync_copy(el_idx.at[pl.ds(w * EPW, EPW)], ia)
    pltpu.sync_copy(el_idx.at[pl.ds(ELBL + w * EPW, EPW)], ib)

    sems = ((sa0, sb0), (sa1, sb1))

    def fire(half, slot):
        roff = slot * HC
        pltpu.async_copy(ot.at[ia.at[pl.ds(half * HC, HC)]],
                         a_v.at[pl.ds(roff, HC)], sems[slot][0])
        pltpu.async_copy(ot.at[ib.at[pl.ds(half * HC, HC)]],
                         b_v.at[pl.ds(roff, HC)], sems[slot][1])

    def wait(half, slot):
        roff = slot * HC
        pltpu.make_async_copy(ot.at[ia.at[pl.ds(half * HC, HC)]],
                              a_v.at[pl.ds(roff, HC)], sems[slot][0]).wait()
        pltpu.make_async_copy(ot.at[ib.at[pl.ds(half * HC, HC)]],
                              b_v.at[pl.ds(roff, HC)], sems[slot][1]).wait()

    def compute(slot):
        roff = slot * HC

        def prow(r, carry2):
            acc = a_v[roff + r, pl.ds(0, 16)] * b_v[roff + r, pl.ds(0, 16)]
            for h in range(1, HG):
                acc = acc + (a_v[roff + r, pl.ds(h * 16, 16)]
                             * b_v[roff + r, pl.ds(h * 16, 16)])
            p_v[pl.ds((roff + r) * 16, 16)] = acc
            return carry2

        lax.fori_loop(0, HC, prow, 0)

    fire(0, 0)

    def chunk(j, carry):
        base = w * EPW + j * CC
        fire(2 * j + 1, 1)
        wait(2 * j, 0)
        compute(0)

        @pl.when(j + 1 < NCC)
        def _():
            fire(2 * j + 2, 0)

        wait(2 * j + 1, 1)
        compute(1)
        pltpu.sync_copy(z_v, o_sp.at[pl.ds(s * CC, CC)])
        pltpu.sync_copy(p_v, o_sp.at[idx_p], add=True)
        pltpu.sync_copy(o_sp.at[pl.ds(s * CC, CC)], out.at[pl.ds(base, CC)])
        return carry

    lax.fori_loop(0, NCC, chunk, 0)


_cls = pl.kernel(
    _cls_body,
    mesh=_mesh,
    out_type=jax.ShapeDtypeStruct((ELBL,), jnp.float32),
    scratch_types=[
        pltpu.VMEM((EPW,), jnp.int32),
        pltpu.VMEM((EPW,), jnp.int32),
        pltpu.VMEM((CC, H), jnp.float32),
        pltpu.VMEM((CC, H), jnp.float32),
        pltpu.VMEM((CC * 16,), jnp.float32),
        pltpu.VMEM((CC * 16,), jnp.int32),
        pltpu.VMEM((CC,), jnp.float32),
        pltpu.VMEM_SHARED((NSUB * CC,), jnp.float32),
        pltpu.SemaphoreType.DMA,
        pltpu.SemaphoreType.DMA,
        pltpu.SemaphoreType.DMA,
        pltpu.SemaphoreType.DMA,
    ],
)


def kernel(node_id_source, node_id_target, edge_index_s2t, edge_index_t2s,
           edge_label_index, src_emb, tgt_emb,
           W1_nbr_s2t, W1_root_tgt, b1_tgt, W1_nbr_t2s, W1_root_src, b1_src,
           W2_nbr_s2t, W2_root_tgt, b2_tgt, W2_nbr_t2s, W2_root_src, b2_src):
    i32 = jnp.int32
    T1 = jnp.concatenate([src_emb, tgt_emb], axis=0)
    src_all = jnp.concatenate([edge_index_s2t[0].astype(i32),
                               edge_index_t2s[0].astype(i32) + N])
    dst_all = jnp.concatenate([edge_index_s2t[1].astype(i32),
                               edge_index_t2s[1].astype(i32)])
    el_all = jnp.concatenate([edge_label_index[0].astype(i32),
                              edge_label_index[1].astype(i32) + N])

    mean1, inv_deg = _agg1(T1, src_all, dst_all)
    Wn1 = jnp.stack([W1_nbr_s2t, W1_nbr_t2s])
    Wr1 = jnp.stack([W1_root_tgt, W1_root_src])
    b1 = jnp.stack([b1_tgt, b1_src])[:, None, :]
    T2 = _transform_relu(mean1, T1.reshape(2, N, H), Wn1, Wr1, b1)

    mean2 = _agg2(T2.reshape(2 * N, H), src_all, dst_all, inv_deg)
    Wn2 = jnp.stack([W2_nbr_s2t, W2_nbr_t2s])
    Wr2 = jnp.stack([W2_root_tgt, W2_root_src])
    b2 = jnp.stack([b2_tgt, b2_src])[:, None, :]
    OT = _transform_lin(mean2, T2, Wn2, Wr2, b2)

    return _cls(OT.reshape(2 * N, H), el_all)


# speedup vs baseline: 1.1808x; 1.1808x over previous
import functools

import jax
import jax.numpy as jnp
from jax import lax
from jax.experimental import pallas as pl
from jax.experimental.pallas import tpu as pltpu
from jax.experimental.pallas import tpu_sc as plsc

H = 128
HG = H // 16
N = 10000
EDGES = 320000
ELBL = 131072
NCORES = 2
NSUB = 16
NW = NCORES * NSUB

EC = EDGES // NSUB
CH = 80
IBLK = 10
SBLK = CH * IBLK
NSB = EC // SBLK
MCH = 80
NMCH = N // MCH

DCH = 800
DNCHUNK = EC // DCH
DMC = 400
DNMC = N // DMC

CC = 256
EPW = ELBL // NW
NCC = EPW // CC

_mesh = plsc.VectorSubcoreMesh(core_axis_name="c", subcore_axis_name="s")


def _agg_body(want_deg, *refs):
    if want_deg:
        (table, src_idx, dst_idx, mean_out, inv_out,
         acc, deg_sp, idxsA, idxdA, idxsB, idxdB, ones_v, d_v, rows0, rows1,
         gsem0, gsem1, ssem0, ssem1) = refs
    else:
        (table, src_idx, dst_idx, inv_deg, mean_out,
         acc, idxsA, idxdA, idxsB, idxdB, d_v, rows0, rows1,
         gsem0, gsem1, ssem0, ssem1) = refs
    c = lax.axis_index("c")
    s = lax.axis_index("s")
    nmine = (NMCH - s + NSUB - 1) // NSUB
    z16v = jnp.zeros((16,), jnp.float32)

    def fill0(r, carry):
        for h in range(HG):
            rows0[r, pl.ds(h * 16, 16)] = z16v
        return carry

    lax.fori_loop(0, MCH, fill0, 0)
    if want_deg:
        one16 = jnp.full((16,), 1.0, jnp.float32)

        def fillz1(g, carry):
            ones_v[pl.ds(g * 16, 16)] = z16v
            return carry

        lax.fori_loop(0, SBLK // 16, fillz1, 0)

    def zchunk(k, carry):
        row0 = (s + k * NSUB) * MCH
        pltpu.sync_copy(rows0, acc.at[pl.ds(row0, MCH)])
        if want_deg:
            pltpu.sync_copy(ones_v.at[pl.ds(0, MCH)], deg_sp.at[pl.ds(row0, MCH)])
        return carry

    lax.fori_loop(0, nmine, zchunk, 0)
    if want_deg:
        def fillo(g, carry):
            ones_v[pl.ds(g * 16, 16)] = one16
            return carry

        lax.fori_loop(0, SBLK // 16, fillo, 0)
    plsc.subcore_barrier()

    rowbufs = (rows0, rows1)
    gsems = (gsem0, gsem1)
    ssems = (ssem0, ssem1)
    idxbufs = ((idxsA, idxdA), (idxsB, idxdB))
    isems = (ssem0, ssem1)

    def fire_idx(j, pair):
        base = c * EDGES + s * EC + j * SBLK
        pltpu.async_copy(src_idx.at[pl.ds(base, SBLK)], idxbufs[pair][0],
                         isems[pair])
        pltpu.async_copy(dst_idx.at[pl.ds(base, SBLK)], idxbufs[pair][1],
                         isems[pair])

    def wait_idx(j, pair):
        base = c * EDGES + s * EC + j * SBLK
        pltpu.make_async_copy(src_idx.at[pl.ds(base, SBLK)], idxbufs[pair][0],
                              isems[pair]).wait()
        pltpu.make_async_copy(dst_idx.at[pl.ds(base, SBLK)], idxbufs[pair][1],
                              isems[pair]).wait()

    def ring(pair):
        idxs, idxd = idxbufs[pair]
        if want_deg:
            pltpu.sync_copy(ones_v, deg_sp.at[idxd], add=True)
        pltpu.async_copy(table.at[idxs.at[pl.ds(0, CH)]], rows0, gsems[0])
        for i in range(IBLK):
            b = i % 2
            if i + 1 < IBLK:
                nb = (i + 1) % 2
                pltpu.async_copy(
                    table.at[idxs.at[pl.ds((i + 1) * CH, CH)]],
                    rowbufs[nb], gsems[nb])
            pltpu.make_async_copy(
                table.at[idxs.at[pl.ds(i * CH, CH)]],
                rowbufs[b], gsems[b]).wait()
            pltpu.sync_copy(rowbufs[b], acc.at[idxd.at[pl.ds(i * CH, CH)]],
                            add=True)

    fire_idx(0, 0)

    def sbpair(j2, carry):
        fire_idx(2 * j2 + 1, 1)
        wait_idx(2 * j2, 0)
        ring(0)
        fire_idx(2 * j2 + 2, 0)
        wait_idx(2 * j2 + 1, 1)
        ring(1)
        return carry

    lax.fori_loop(0, (NSB - 1) // 2, sbpair, 0)
    wait_idx(NSB - 1, 0)
    ring(0)
    plsc.subcore_barrier()

    def mchunk(k, carry):
        row0 = (s + k * NSUB) * MCH
        pltpu.sync_copy(acc.at[pl.ds(row0, MCH)], rows0)
        if want_deg:
            pltpu.sync_copy(deg_sp.at[pl.ds(row0, MCH)], d_v)

            def igrp(g, carry2):
                v = d_v[pl.ds(g * 16, 16)]
                d_v[pl.ds(g * 16, 16)] = 1.0 / jnp.maximum(v, 1.0)
                return carry2

            lax.fori_loop(0, MCH // 16, igrp, 0)
            pltpu.sync_copy(d_v, inv_out.at[pl.ds(c * N + row0, MCH)])
        else:
            pltpu.sync_copy(inv_deg.at[pl.ds(c * N + row0, MCH)], d_v)

        def mgrp(g, carry2):
            invv = d_v[pl.ds(g * 16, 16)]
            for i in range(16):
                inv = invv[i]
                for h in range(HG):
                    rows0[g * 16 + i, pl.ds(h * 16, 16)] = (
                        rows0[g * 16 + i, pl.ds(h * 16, 16)] * inv)
            return carry2

        lax.fori_loop(0, MCH // 16, mgrp, 0)
        pltpu.sync_copy(rows0, mean_out.at[c, pl.ds(row0, MCH)])
        return carry

    lax.fori_loop(0, nmine, mchunk, 0)


_agg1 = pl.kernel(
    functools.partial(_agg_body, True),
    mesh=_mesh,
    out_type=(jax.ShapeDtypeStruct((NCORES, N, H), jnp.float32),
              jax.ShapeDtypeStruct((NCORES * N,), jnp.float32)),
    scratch_types=[
        pltpu.VMEM_SHARED((N, H), jnp.float32),
        pltpu.VMEM_SHARED((N,), jnp.float32),
        pltpu.VMEM((SBLK,), jnp.int32),
        pltpu.VMEM((SBLK,), jnp.int32),
        pltpu.VMEM((SBLK,), jnp.int32),
        pltpu.VMEM((SBLK,), jnp.int32),
        pltpu.VMEM((SBLK,), jnp.float32),
        pltpu.VMEM((MCH,), jnp.float32),
        pltpu.VMEM((CH, H), jnp.float32),
        pltpu.VMEM((CH, H), jnp.float32),
        pltpu.SemaphoreType.DMA,
        pltpu.SemaphoreType.DMA,
        pltpu.SemaphoreType.DMA,
        pltpu.SemaphoreType.DMA,
    ],
)

_agg2 = pl.kernel(
    functools.partial(_agg_body, False),
    mesh=_mesh,
    out_type=jax.ShapeDtypeStruct((NCORES, N, H), jnp.float32),
    scratch_types=[
        pltpu.VMEM_SHARED((N, H), jnp.float32),
        pltpu.VMEM((SBLK,), jnp.int32),
        pltpu.VMEM((SBLK,), jnp.int32),
        pltpu.VMEM((SBLK,), jnp.int32),
        pltpu.VMEM((SBLK,), jnp.int32),
        pltpu.VMEM((MCH,), jnp.float32),
        pltpu.VMEM((CH, H), jnp.float32),
        pltpu.VMEM((CH, H), jnp.float32),
        pltpu.SemaphoreType.DMA,
        pltpu.SemaphoreType.DMA,
        pltpu.SemaphoreType.DMA,
        pltpu.SemaphoreType.DMA,
    ],
)


def _transform_body(relu, mean_ref, root_ref, wn_ref, wr_ref, b_ref, out_ref):
    o = (jnp.dot(mean_ref[0], wn_ref[0], preferred_element_type=jnp.float32)
         + jnp.dot(root_ref[0], wr_ref[0], preferred_element_type=jnp.float32)
         + b_ref[0])
    if relu:
        o = jnp.maximum(o, 0.0)
    out_ref[0] = o


def _make_transform(relu):
    RB = 1000
    return pl.pallas_call(
        functools.partial(_transform_body, relu),
        grid=(2, N // RB),
        in_specs=[
            pl.BlockSpec((1, RB, H), lambda j, i: (j, i, 0)),
            pl.BlockSpec((1, RB, H), lambda j, i: (1 - j, i, 0)),
            pl.BlockSpec((1, H, H), lambda j, i: (j, 0, 0)),
            pl.BlockSpec((1, H, H), lambda j, i: (j, 0, 0)),
            pl.BlockSpec((1, 1, H), lambda j, i: (j, 0, 0)),
        ],
        out_specs=pl.BlockSpec((1, RB, H), lambda j, i: (1 - j, i, 0)),
        out_shape=jax.ShapeDtypeStruct((2, N, H), jnp.float32),
    )


_transform_relu = _make_transform(True)
_transform_lin = _make_transform(False)


HC = CC // 2


def _cls_body(ot, el_idx, out, ia, ib, a_v, b_v, p_v, idx_p, z_v, o_sp,
              sa0, sb0, sa1, sb1):
    c = lax.axis_index("c")
    s = lax.axis_index("s")
    w = s * NCORES + c
    zero16 = jnp.zeros((16,), jnp.float32)

    def fillidx(g, carry):
        idx_p[pl.ds(g * 16, 16)] = jnp.full((16,), s * CC + g, jnp.int32)
        return carry

    lax.fori_loop(0, CC, fillidx, 0)

    def fillz(g, carry):
        z_v[pl.ds(g * 16, 16)] = zero16
        return carry

    lax.fori_loop(0, CC // 16, fillz, 0)

    pltpu.sync_copy(el_idx.at[pl.ds(w * EPW, EPW)], ia)
    pltpu.sync_copy(el_idx.at[pl.ds(ELBL + w * EPW, EPW)], ib)

    sems = ((sa0, sb0), (sa1, sb1))

    def fire(half, slot):
        roff = slot * HC
        pltpu.async_copy(ot.at[ia.at[pl.ds(half * HC, HC)]],
                         a_v.at[pl.ds(roff, HC)], sems[slot][0])
        pltpu.async_copy(ot.at[ib.at[pl.ds(half * HC, HC)]],
                         b_v.at[pl.ds(roff, HC)], sems[slot][1])

    def wait(half, slot):
        roff = slot * HC
        pltpu.make_async_copy(ot.at[ia.at[pl.ds(half * HC, HC)]],
                              a_v.at[pl.ds(roff, HC)], sems[slot][0]).wait()
        pltpu.make_async_copy(ot.at[ib.at[pl.ds(half * HC, HC)]],
                              b_v.at[pl.ds(roff, HC)], sems[slot][1]).wait()

    def compute(slot):
        roff = slot * HC

        def prow(r, carry2):
            acc = a_v[roff + r, pl.ds(0, 16)] * b_v[roff + r, pl.ds(0, 16)]
            for h in range(1, HG):
                acc = acc + (a_v[roff + r, pl.ds(h * 16, 16)]
                             * b_v[roff + r, pl.ds(h * 16, 16)])
            p_v[pl.ds((roff + r) * 16, 16)] = acc
            return carry2

        lax.fori_loop(0, HC, prow, 0)

    fire(0, 0)

    def chunk(j, carry):
        base = w * EPW + j * CC
        fire(2 * j + 1, 1)
        wait(2 * j, 0)
        compute(0)

        @pl.when(j + 1 < NCC)
        def _():
            fire(2 * j + 2, 0)

        wait(2 * j + 1, 1)
        compute(1)
        pltpu.sync_copy(z_v, o_sp.at[pl.ds(s * CC, CC)])
        pltpu.sync_copy(p_v, o_sp.at[idx_p], add=True)
        pltpu.sync_copy(o_sp.at[pl.ds(s * CC, CC)], out.at[pl.ds(base, CC)])
        return carry

    lax.fori_loop(0, NCC, chunk, 0)


_cls = pl.kernel(
    _cls_body,
    mesh=_mesh,
    out_type=jax.ShapeDtypeStruct((ELBL,), jnp.float32),
    scratch_types=[
        pltpu.VMEM((EPW,), jnp.int32),
        pltpu.VMEM((EPW,), jnp.int32),
        pltpu.VMEM((CC, H), jnp.float32),
        pltpu.VMEM((CC, H), jnp.float32),
        pltpu.VMEM((CC * 16,), jnp.float32),
        pltpu.VMEM((CC * 16,), jnp.int32),
        pltpu.VMEM((CC,), jnp.float32),
        pltpu.VMEM_SHARED((NSUB * CC,), jnp.float32),
        pltpu.SemaphoreType.DMA,
        pltpu.SemaphoreType.DMA,
        pltpu.SemaphoreType.DMA,
        pltpu.SemaphoreType.DMA,
    ],
)


def kernel(node_id_source, node_id_target, edge_index_s2t, edge_index_t2s,
           edge_label_index, src_emb, tgt_emb,
           W1_nbr_s2t, W1_root_tgt, b1_tgt, W1_nbr_t2s, W1_root_src, b1_src,
           W2_nbr_s2t, W2_root_tgt, b2_tgt, W2_nbr_t2s, W2_root_src, b2_src):
    i32 = jnp.int32
    T1 = jnp.concatenate([src_emb, tgt_emb], axis=0)
    src_all = jnp.concatenate([edge_index_s2t[0].astype(i32),
                               edge_index_t2s[0].astype(i32) + N])
    dst_all = jnp.concatenate([edge_index_s2t[1].astype(i32),
                               edge_index_t2s[1].astype(i32)])
    el_all = jnp.concatenate([edge_label_index[0].astype(i32),
                              edge_label_index[1].astype(i32) + N])

    mean1, inv_deg = _agg1(T1, src_all, dst_all)
    Wn1 = jnp.stack([W1_nbr_s2t, W1_nbr_t2s])
    Wr1 = jnp.stack([W1_root_tgt, W1_root_src])
    b1 = jnp.stack([b1_tgt, b1_src])[:, None, :]
    T2 = _transform_relu(mean1, T1.reshape(2, N, H), Wn1, Wr1, b1)

    mean2 = _agg2(T2.reshape(2 * N, H), src_all, dst_all, inv_deg)
    Wn2 = jnp.stack([W2_nbr_s2t, W2_nbr_t2s])
    Wr2 = jnp.stack([W2_root_tgt, W2_root_src])
    b2 = jnp.stack([b2_tgt, b2_src])[:, None, :]
    OT = _transform_lin(mean2, T2, Wn2, Wr2, b2)

    return _cls(OT.reshape(2 * N, H), el_all)
